# baseline (speedup 1.0000x reference)
import functools

import jax
import jax.numpy as jnp
from jax import lax
from jax.experimental import pallas as pl
from jax.experimental.pallas import tpu as pltpu
from jax.experimental.pallas import tpu_sc as plsc

N = 10000
E = 320000
NC, NS, L = 2, 16, 16
NW = NC * NS
EPW = E // NW
C = 256
NCH = EPW // C
TAIL = EPW - NCH * C
NP = 10240
RPT = NP // NS
F = 64
RB = 1024
_mesh = plsc.VectorSubcoreMesh(
    core_axis_name="c", subcore_axis_name="s", num_cores=NC, num_subcores=NS)


def _sc_degree(dst, zeros_n):
    @functools.partial(
        pl.kernel,
        out_type=jax.ShapeDtypeStruct((NW, NP), jnp.float32),
        mesh=_mesh,
        scratch_types=[
            pltpu.VMEM((EPW,), jnp.int32),
            pltpu.VMEM((NP,), jnp.float32),
        ],
        compiler_params=pltpu.CompilerParams(needs_layout_passes=False),
    )
    def body(dst_hbm, zeros_hbm, out_hbm, idx_v, hist_v):
        cid = lax.axis_index("c")
        sid = lax.axis_index("s")
        wid = sid * NC + cid
        pltpu.sync_copy(zeros_hbm, hist_v)
        pltpu.sync_copy(dst_hbm.at[pl.ds(wid * EPW, EPW)], idx_v)
        ones = jnp.ones((L,), jnp.float32)

        def step(j, carry):
            idx = idx_v[pl.ds(j * L, L)]
            plsc.addupdate_scatter(hist_v, [idx], ones)
            return carry

        lax.fori_loop(0, EPW // L, step, 0)
        pltpu.sync_copy(hist_v, out_hbm.at[wid])

    return body(dst, zeros_n)


def _sc_aggregate(yw, src, dst, zeros_t):
    @functools.partial(
        pl.kernel,
        out_type=jax.ShapeDtypeStruct((NC, NP, F), jnp.float32),
        mesh=_mesh,
        scratch_types=[
            pltpu.VMEM((EPW,), jnp.int32),
            pltpu.VMEM((EPW,), jnp.int32),
            pltpu.VMEM((C, F), jnp.float32),
            pltpu.VMEM((C, F), jnp.float32),
            pltpu.VMEM((C, F), jnp.float32),
            pltpu.VMEM((TAIL, F), jnp.float32),
            pltpu.VMEM_SHARED((NP, F), jnp.float32),
            pltpu.SemaphoreType.DMA,
            pltpu.SemaphoreType.DMA,
            pltpu.SemaphoreType.DMA,
            pltpu.SemaphoreType.DMA,
            pltpu.SemaphoreType.DMA,
            pltpu.SemaphoreType.DMA,
        ],
        compiler_params=pltpu.CompilerParams(
            needs_layout_passes=False, use_tc_tiling_on_sc=False),
    )
    def body(yw_hbm, src_hbm, dst_hbm, zeros_hbm, out_hbm, sidx, didx,
             r0, r1, r2, rt, acc, g0, g1, g2, s0, s1, s2):
        cid = lax.axis_index("c")
        sid = lax.axis_index("s")
        wid = sid * NC + cid
        row_b = (r0, r1, r2)
        gs = (g0, g1, g2)
        ss = (s0, s1, s2)
        pltpu.sync_copy(zeros_hbm, acc.at[pl.ds(sid * RPT, RPT)])
        plsc.subcore_barrier()

        base = wid * EPW
        pltpu.sync_copy(src_hbm.at[pl.ds(base, EPW)], sidx)
        pltpu.sync_copy(dst_hbm.at[pl.ds(base, EPW)], didx)

        def start_gather(m, rb, gb):
            pltpu.async_copy(yw_hbm.at[sidx.at[pl.ds(m * C, C)]], rb, gb)

        def wait_gather(m, rb, gb):
            pltpu.make_async_copy(yw_hbm.at[sidx.at[pl.ds(m * C, C)]], rb, gb).wait()

        def dchunk(m):
            return didx.at[pl.ds(m * C, C)]

        start_gather(0, r0, g0)
        start_gather(1, r1, g1)

        def triple(i, carry):
            for b in range(3):
                k = i * 3 + b
                bm1 = (b + 2) % 3
                wait_gather(k, row_b[b], gs[b])
                pltpu.async_copy(row_b[b], acc.at[dchunk(k)], ss[b], add=True)

                @pl.when(k >= 1)
                def _wait_prev_scatter():
                    pltpu.make_async_copy(
                        row_b[bm1], acc.at[dchunk(k - 1)], ss[bm1]).wait()

                @pl.when(k + 2 < NCH)
                def _prefetch():
                    start_gather(k + 2, row_b[bm1], gs[bm1])

            return carry

        lax.fori_loop(0, NCH // 3, triple, 0)
        _blast = (NCH - 1) % 3
        pltpu.make_async_copy(row_b[_blast], acc.at[dchunk(NCH - 1)], ss[_blast]).wait()
        pltpu.async_copy(yw_hbm.at[sidx.at[pl.ds(NCH * C, TAIL)]], rt, g0).wait()
        pltpu.sync_copy(rt, acc.at[didx.at[pl.ds(NCH * C, TAIL)]], add=True)
        plsc.subcore_barrier()
        pltpu.sync_copy(acc.at[pl.ds(sid * RPT, RPT)],
                        out_hbm.at[cid, pl.ds(sid * RPT, RPT)])

    return body(yw, src, dst, zeros_t)


def _tc_prep(x_p, W1, degp):
    def body(x_ref, w_ref, degp_ref, dis_ref, yw_ref):
        deg = jnp.sum(degp_ref[...], axis=0) + 1.0
        dis = lax.rsqrt(deg)[:, None]
        xw = jnp.dot(x_ref[...], w_ref[...], preferred_element_type=jnp.float32)
        dis_ref[...] = dis
        yw_ref[...] = xw * dis

    return pl.pallas_call(
        body,
        grid=(NP // RB,),
        in_specs=[
            pl.BlockSpec((RB, 128), lambda i: (i, 0)),
            pl.BlockSpec((128, F), lambda i: (0, 0)),
            pl.BlockSpec((NW, RB), lambda i: (0, i)),
        ],
        out_specs=[
            pl.BlockSpec((RB, 1), lambda i: (i, 0)),
            pl.BlockSpec((RB, F), lambda i: (i, 0)),
        ],
        out_shape=[
            jax.ShapeDtypeStruct((NP, 1), jnp.float32),
            jax.ShapeDtypeStruct((NP, F), jnp.float32),
        ],
    )(x_p, W1, degp)


def _tc_layer(p, yw_prev, dis, b, W_next):
    def body(p_ref, yw_ref, dis_ref, b_ref, w_ref, out_ref):
        dis = dis_ref[...]
        agg = jnp.sum(p_ref[...], axis=0) + yw_ref[...]
        h = jnp.maximum(agg * dis + b_ref[...], 0.0)
        out_ref[...] = jnp.dot(h, w_ref[...],
                               preferred_element_type=jnp.float32) * dis

    return pl.pallas_call(
        body,
        grid=(NP // RB,),
        in_specs=[
            pl.BlockSpec((NC, RB, F), lambda i: (0, i, 0)),
            pl.BlockSpec((RB, F), lambda i: (i, 0)),
            pl.BlockSpec((RB, 1), lambda i: (i, 0)),
            pl.BlockSpec((1, F), lambda i: (0, 0)),
            pl.BlockSpec((F, F), lambda i: (0, 0)),
        ],
        out_specs=pl.BlockSpec((RB, F), lambda i: (i, 0)),
        out_shape=jax.ShapeDtypeStruct((NP, F), jnp.float32),
    )(p, yw_prev, dis, b.reshape(1, F), W_next)


def _tc_final(p, yw_prev, dis, b, Wl, bl):
    def body(p_ref, yw_ref, dis_ref, b_ref, wl_ref, bl_ref, h_ref, z_ref):
        dis = dis_ref[...]
        agg = jnp.sum(p_ref[...], axis=0) + yw_ref[...]
        h = jnp.maximum(agg * dis + b_ref[...], 0.0)
        h_ref[...] = h
        z_ref[...] = jnp.dot(h, wl_ref[...],
                             preferred_element_type=jnp.float32) + bl_ref[...]

    return pl.pallas_call(
        body,
        grid=(NP // RB,),
        in_specs=[
            pl.BlockSpec((NC, RB, F), lambda i: (0, i, 0)),
            pl.BlockSpec((RB, F), lambda i: (i, 0)),
            pl.BlockSpec((RB, 1), lambda i: (i, 0)),
            pl.BlockSpec((1, F), lambda i: (0, 0)),
            pl.BlockSpec((F, 4), lambda i: (0, 0)),
            pl.BlockSpec((1, 4), lambda i: (0, 0)),
        ],
        out_specs=[
            pl.BlockSpec((RB, F), lambda i: (i, 0)),
            pl.BlockSpec((RB, 4), lambda i: (i, 0)),
        ],
        out_shape=[
            jax.ShapeDtypeStruct((NP, F), jnp.float32),
            jax.ShapeDtypeStruct((NP, 4), jnp.float32),
        ],
    )(p, yw_prev, dis, b.reshape(1, F), Wl, bl.reshape(1, 4))


def kernel(x, W1, b1, W2, b2, Wl, bl, edges):
    src = edges[0]
    dst = edges[1]
    zeros_n = jnp.zeros((NP,), jnp.float32)
    zeros_t = jnp.zeros((RPT, F), jnp.float32)
    x_p = jnp.concatenate([x, jnp.zeros((NP - N, 128), jnp.float32)], axis=0)

    degp = _sc_degree(dst, zeros_n)
    dis, yw1 = _tc_prep(x_p, W1, degp)
    p1 = _sc_aggregate(yw1, src, dst, zeros_t)
    yw2 = _tc_layer(p1, yw1, dis, b1, W2)
    p2 = _sc_aggregate(yw2, src, dst, zeros_t)
    h_p, z_p = _tc_final(p2, yw2, dis, b2, Wl, bl)
    return (h_p[:N], z_p[:N])


# speedup vs baseline: 1.0109x; 1.0109x over previous
import functools

import jax
import jax.numpy as jnp
from jax import lax
from jax.experimental import pallas as pl
from jax.experimental.pallas import tpu as pltpu
from jax.experimental.pallas import tpu_sc as plsc

N = 10000
E = 320000
NC, NS, L = 2, 16, 16
NW = NC * NS
EPW = E // NW
C = 128
NCH = EPW // C
TAIL = EPW - NCH * C
NP = 10240
RPT = NP // NS
F = 64
RB = 1024
_mesh = plsc.VectorSubcoreMesh(
    core_axis_name="c", subcore_axis_name="s", num_cores=NC, num_subcores=NS)


def _sc_degree(dst, zeros_n):
    @functools.partial(
        pl.kernel,
        out_type=jax.ShapeDtypeStruct((NW, NP), jnp.float32),
        mesh=_mesh,
        scratch_types=[
            pltpu.VMEM((EPW,), jnp.int32),
            pltpu.VMEM((NP,), jnp.float32),
        ],
        compiler_params=pltpu.CompilerParams(needs_layout_passes=False),
    )
    def body(dst_hbm, zeros_hbm, out_hbm, idx_v, hist_v):
        cid = lax.axis_index("c")
        sid = lax.axis_index("s")
        wid = sid * NC + cid
        pltpu.sync_copy(zeros_hbm, hist_v)
        pltpu.sync_copy(dst_hbm.at[pl.ds(wid * EPW, EPW)], idx_v)
        ones = jnp.ones((L,), jnp.float32)

        def step(j, carry):
            for u in range(5):
                idx = idx_v[pl.ds((j * 5 + u) * L, L)]
                plsc.addupdate_scatter(hist_v, [idx], ones)
            return carry

        lax.fori_loop(0, EPW // (5 * L), step, 0)
        pltpu.sync_copy(hist_v, out_hbm.at[wid])

    return body(dst, zeros_n)


def _sc_aggregate(yw, src, dst, zeros_t):
    @functools.partial(
        pl.kernel,
        out_type=jax.ShapeDtypeStruct((NC, NP, F), jnp.float32),
        mesh=_mesh,
        scratch_types=[
            pltpu.VMEM((EPW,), jnp.int32),
            pltpu.VMEM((EPW,), jnp.int32),
            pltpu.VMEM((C, F), jnp.float32),
            pltpu.VMEM((C, F), jnp.float32),
            pltpu.VMEM((C, F), jnp.float32),
            pltpu.VMEM((TAIL, F), jnp.float32),
            pltpu.VMEM_SHARED((NP, F), jnp.float32),
            pltpu.SemaphoreType.DMA,
            pltpu.SemaphoreType.DMA,
            pltpu.SemaphoreType.DMA,
            pltpu.SemaphoreType.DMA,
            pltpu.SemaphoreType.DMA,
            pltpu.SemaphoreType.DMA,
        ],
        compiler_params=pltpu.CompilerParams(
            needs_layout_passes=False, use_tc_tiling_on_sc=False),
    )
    def body(yw_hbm, src_hbm, dst_hbm, zeros_hbm, out_hbm, sidx, didx,
             r0, r1, r2, rt, acc, g0, g1, g2, s0, s1, s2):
        cid = lax.axis_index("c")
        sid = lax.axis_index("s")
        wid = sid * NC + cid
        row_b = (r0, r1, r2)
        gs = (g0, g1, g2)
        ss = (s0, s1, s2)
        pltpu.sync_copy(zeros_hbm, acc.at[pl.ds(sid * RPT, RPT)])
        plsc.subcore_barrier()

        base = wid * EPW
        pltpu.sync_copy(src_hbm.at[pl.ds(base, EPW)], sidx)
        pltpu.sync_copy(dst_hbm.at[pl.ds(base, EPW)], didx)

        def start_gather(m, rb, gb):
            pltpu.async_copy(yw_hbm.at[sidx.at[pl.ds(m * C, C)]], rb, gb)

        def wait_gather(m, rb, gb):
            pltpu.make_async_copy(yw_hbm.at[sidx.at[pl.ds(m * C, C)]], rb, gb).wait()

        def dchunk(m):
            return didx.at[pl.ds(m * C, C)]

        start_gather(0, r0, g0)
        start_gather(1, r1, g1)

        def triple(i, carry):
            for b in range(3):
                k = i * 3 + b
                bm1 = (b + 2) % 3
                wait_gather(k, row_b[b], gs[b])
                pltpu.async_copy(row_b[b], acc.at[dchunk(k)], ss[b], add=True)

                @pl.when(k >= 1)
                def _wait_prev_scatter():
                    pltpu.make_async_copy(
                        row_b[bm1], acc.at[dchunk(k - 1)], ss[bm1]).wait()

                @pl.when(k + 2 < NCH)
                def _prefetch():
                    start_gather(k + 2, row_b[bm1], gs[bm1])

            return carry

        lax.fori_loop(0, NCH // 3, triple, 0)
        _blast = (NCH - 1) % 3
        pltpu.make_async_copy(row_b[_blast], acc.at[dchunk(NCH - 1)], ss[_blast]).wait()
        pltpu.async_copy(yw_hbm.at[sidx.at[pl.ds(NCH * C, TAIL)]], rt, g0).wait()
        pltpu.sync_copy(rt, acc.at[didx.at[pl.ds(NCH * C, TAIL)]], add=True)
        plsc.subcore_barrier()
        pltpu.sync_copy(acc.at[pl.ds(sid * RPT, RPT)],
                        out_hbm.at[cid, pl.ds(sid * RPT, RPT)])

    return body(yw, src, dst, zeros_t)


def _tc_prep(x_p, W1, degp):
    def body(x_ref, w_ref, degp_ref, dis_ref, yw_ref):
        deg = jnp.sum(degp_ref[...], axis=0) + 1.0
        dis = lax.rsqrt(deg)[:, None]
        xw = jnp.dot(x_ref[...], w_ref[...], preferred_element_type=jnp.float32)
        dis_ref[...] = dis
        yw_ref[...] = xw * dis

    return pl.pallas_call(
        body,
        grid=(NP // RB,),
        in_specs=[
            pl.BlockSpec((RB, 128), lambda i: (i, 0)),
            pl.BlockSpec((128, F), lambda i: (0, 0)),
            pl.BlockSpec((NW, RB), lambda i: (0, i)),
        ],
        out_specs=[
            pl.BlockSpec((RB, 1), lambda i: (i, 0)),
            pl.BlockSpec((RB, F), lambda i: (i, 0)),
        ],
        out_shape=[
            jax.ShapeDtypeStruct((NP, 1), jnp.float32),
            jax.ShapeDtypeStruct((NP, F), jnp.float32),
        ],
    )(x_p, W1, degp)


def _tc_layer(p, yw_prev, dis, b, W_next):
    def body(p_ref, yw_ref, dis_ref, b_ref, w_ref, out_ref):
        dis = dis_ref[...]
        agg = jnp.sum(p_ref[...], axis=0) + yw_ref[...]
        h = jnp.maximum(agg * dis + b_ref[...], 0.0)
        out_ref[...] = jnp.dot(h, w_ref[...],
                               preferred_element_type=jnp.float32) * dis

    return pl.pallas_call(
        body,
        grid=(NP // RB,),
        in_specs=[
            pl.BlockSpec((NC, RB, F), lambda i: (0, i, 0)),
            pl.BlockSpec((RB, F), lambda i: (i, 0)),
            pl.BlockSpec((RB, 1), lambda i: (i, 0)),
            pl.BlockSpec((1, F), lambda i: (0, 0)),
            pl.BlockSpec((F, F), lambda i: (0, 0)),
        ],
        out_specs=pl.BlockSpec((RB, F), lambda i: (i, 0)),
        out_shape=jax.ShapeDtypeStruct((NP, F), jnp.float32),
    )(p, yw_prev, dis, b.reshape(1, F), W_next)


def _tc_final(p, yw_prev, dis, b, Wl, bl):
    def body(p_ref, yw_ref, dis_ref, b_ref, wl_ref, bl_ref, h_ref, z_ref):
        dis = dis_ref[...]
        agg = jnp.sum(p_ref[...], axis=0) + yw_ref[...]
        h = jnp.maximum(agg * dis + b_ref[...], 0.0)
        h_ref[...] = h
        z_ref[...] = jnp.dot(h, wl_ref[...],
                             preferred_element_type=jnp.float32) + bl_ref[...]

    return pl.pallas_call(
        body,
        grid=(NP // RB,),
        in_specs=[
            pl.BlockSpec((NC, RB, F), lambda i: (0, i, 0)),
            pl.BlockSpec((RB, F), lambda i: (i, 0)),
            pl.BlockSpec((RB, 1), lambda i: (i, 0)),
            pl.BlockSpec((1, F), lambda i: (0, 0)),
            pl.BlockSpec((F, 4), lambda i: (0, 0)),
            pl.BlockSpec((1, 4), lambda i: (0, 0)),
        ],
        out_specs=[
            pl.BlockSpec((RB, F), lambda i: (i, 0)),
            pl.BlockSpec((RB, 4), lambda i: (i, 0)),
        ],
        out_shape=[
            jax.ShapeDtypeStruct((NP, F), jnp.float32),
            jax.ShapeDtypeStruct((NP, 4), jnp.float32),
        ],
    )(p, yw_prev, dis, b.reshape(1, F), Wl, bl.reshape(1, 4))


def kernel(x, W1, b1, W2, b2, Wl, bl, edges):
    src = edges[0]
    dst = edges[1]
    zeros_n = jnp.zeros((NP,), jnp.float32)
    zeros_t = jnp.zeros((RPT, F), jnp.float32)
    x_p = jnp.concatenate([x, jnp.zeros((NP - N, 128), jnp.float32)], axis=0)

    degp = _sc_degree(dst, zeros_n)
    dis, yw1 = _tc_prep(x_p, W1, degp)
    p1 = _sc_aggregate(yw1, src, dst, zeros_t)
    yw2 = _tc_layer(p1, yw1, dis, b1, W2)
    p2 = _sc_aggregate(yw2, src, dst, zeros_t)
    h_p, z_p = _tc_final(p2, yw2, dis, b2, Wl, bl)
    return (h_p[:N], z_p[:N])
